# baseline (speedup 1.0000x reference)
import jax
import jax.numpy as jnp
from jax.experimental import pallas as pl

_B = 16
_IMG = 224
_P = 16
_C = 384
_H = 6
_DH = 64
_DEPTH_EARLY = 4
_DEPTH = 12
_NCLS = 1000
_G = _IMG // _P
_NP = _G * _G
_S = _NP + 1
_SP = 224
_KK = _S // 2
_L = _KK + 1
_LP = 128
_PD = 3 * _P * _P
_MT = 512

_F32 = jnp.float32
_BF16 = jnp.bfloat16


def _mm(a, b):
    return jax.lax.dot_general(a.astype(_BF16), b.astype(_BF16),
                               (((1,), (0,)), ((), ())),
                               preferred_element_type=_F32)


def _mm_t(a, b):
    return jax.lax.dot_general(a.astype(_BF16), b.astype(_BF16),
                               (((1,), (1,)), ((), ())),
                               preferred_element_type=_F32)


def _mm_tt_exact(a, b):
    return jax.lax.dot_general(a, b, (((0,), (0,)), ((), ())),
                               precision=jax.lax.Precision.HIGHEST,
                               preferred_element_type=_F32)


def _sum8tree(acc):
    s = acc[:, 0:8]
    for a in range(1, acc.shape[1] // 8):
        s = s + acc[:, 8 * a:8 * a + 8]
    t = s[:, 0:4] + s[:, 4:8]
    t = t[:, 0:2] + t[:, 2:4]
    return t[:, 0:1] + t[:, 1:2]


def _sum384(x):
    return _sum8tree((x[:, 0:128] + x[:, 128:256]) + x[:, 256:384])


def _ln(x, s, b):
    mu = _sum384(x) * (1.0 / 384.0)
    xc = x - mu
    v = _sum384(xc * xc) * (1.0 / 384.0)
    return xc * jax.lax.rsqrt(v + 1e-6) * s + b


def _softmax(lg):
    m = jnp.max(lg, axis=-1, keepdims=True)
    e = jnp.exp(lg - m)
    return e / _sum8tree(e)


def _full2d(shape):
    return pl.BlockSpec(shape, lambda *_: (0, 0))


def _embed_body(p_ref, w_ref, a_ref, o_ref):
    o_ref[...] = _mm(p_ref[...], w_ref[...]) + a_ref[...]


def _lnqkv_body(x_ref, s_ref, b_ref, w_ref, wb_ref, o_ref):
    xn = _ln(x_ref[...], s_ref[...], b_ref[...])
    o_ref[...] = _mm(xn, w_ref[...]) + wb_ref[...]


def _projmlp1_body(x_ref, o_ref, pw_ref, pb_ref, s_ref, b_ref, w1_ref,
                   b1_ref, x1_ref, h_ref):
    x1 = x_ref[...] + (_mm(o_ref[...], pw_ref[...]) + pb_ref[...])
    x1_ref[...] = x1
    xn2 = _ln(x1, s_ref[...], b_ref[...])
    h_ref[...] = jax.nn.gelu(_mm(xn2, w1_ref[...]) + b1_ref[...])


def _mlp2_body(x1_ref, h_ref, w2_ref, b2_ref, y_ref):
    y_ref[...] = x1_ref[...] + (_mm(h_ref[...], w2_ref[...]) + b2_ref[...])


def _rowtiled(body, m, ins, outs, n_rowtiled_in):
    in_specs = []
    args = []
    for idx, a in enumerate(ins):
        args.append(a)
        if idx < n_rowtiled_in:
            in_specs.append(pl.BlockSpec((_MT, a.shape[1]), lambda g: (g, 0)))
        else:
            in_specs.append(_full2d(a.shape))
    out_specs = [pl.BlockSpec((_MT, n), lambda g: (g, 0)) for n in outs]
    out_shape = [jax.ShapeDtypeStruct((m, n), _F32) for n in outs]
    if len(outs) == 1:
        out_specs, out_shape = out_specs[0], out_shape[0]
    return pl.pallas_call(body, grid=(m // _MT,), in_specs=in_specs,
                          out_specs=out_specs, out_shape=out_shape)(*args)


def _make_attn_body(sp, valid, capture):
    def body(qkv_ref, *out_refs):
        qkv = qkv_ref[...]
        lane = jax.lax.broadcasted_iota(jnp.int32, (sp, sp), 1)
        kmask = lane < valid
        scale = _DH ** -0.5
        outs = []
        cls_rows = []
        for h in range(_H):
            q = qkv[:, h * _DH:(h + 1) * _DH]
            k = qkv[:, _C + h * _DH:_C + (h + 1) * _DH]
            v = qkv[:, 2 * _C + h * _DH:2 * _C + (h + 1) * _DH]
            lg = _mm_t(q, k) * scale
            lg = jnp.where(kmask, lg, -1e30)
            a = _softmax(lg)
            if capture:
                cls_rows.append(a[0:1, :])
            outs.append(_mm(a, v))
        out_refs[0][...] = jnp.concatenate(outs, axis=1)
        if capture:
            out_refs[1][0] = jnp.mean(jnp.concatenate(cls_rows, axis=0),
                                      axis=0, keepdims=True)
    return body


def _attn_call(qkv, sp, valid, capture):
    b = qkv.shape[0] // sp
    out_specs = [pl.BlockSpec((sp, _C), lambda g: (g, 0))]
    out_shape = [jax.ShapeDtypeStruct((b * sp, _C), _F32)]
    if capture:
        out_specs.append(pl.BlockSpec((1, 1, sp), lambda g: (g, 0, 0)))
        out_shape.append(jax.ShapeDtypeStruct((b, 1, sp), _F32))
    res = pl.pallas_call(
        _make_attn_body(sp, valid, capture),
        grid=(b,),
        in_specs=[pl.BlockSpec((sp, 3 * _C), lambda g: (g, 0))],
        out_specs=out_specs,
        out_shape=out_shape,
    )(qkv)
    return res if capture else res[0]


def _block(x, i, qkv_w, qkv_b, proj_w, proj_b, ln1_s, ln1_b, ln2_s, ln2_b,
           mlp_w1, mlp_b1, mlp_w2, mlp_b2, sp, valid, capture):
    m = x.shape[0]
    qkv = _rowtiled(_lnqkv_body, m,
                    [x, ln1_s[i].reshape(1, _C), ln1_b[i].reshape(1, _C),
                     qkv_w[i], qkv_b[i].reshape(1, 3 * _C)],
                    [3 * _C], 1)
    ares = _attn_call(qkv, sp, valid, capture)
    if capture:
        o, cls_attn = ares
    else:
        o, cls_attn = ares, None
    x1, hid = _rowtiled(_projmlp1_body, m,
                        [x, o, proj_w[i], proj_b[i].reshape(1, _C),
                         ln2_s[i].reshape(1, _C), ln2_b[i].reshape(1, _C),
                         mlp_w1[i], mlp_b1[i].reshape(1, 4 * _C)],
                        [_C, 4 * _C], 2)
    y = pl.pallas_call(
        _mlp2_body,
        in_specs=[_full2d((m, _C)), _full2d((m, 4 * _C)),
                  _full2d((4 * _C, _C)), _full2d((1, _C))],
        out_specs=_full2d((m, _C)),
        out_shape=jax.ShapeDtypeStruct((m, _C), _F32),
    )(x1, hid, mlp_w2[i], mlp_b2[i].reshape(1, _C))
    return (y, cls_attn) if capture else y


def _pack_body(x_ref, ca_ref, o_ref):
    x = x_ref[0]
    row = ca_ref[0]
    s_j = jnp.broadcast_to(row, (_SP, _SP))
    ones_row = jnp.ones((1, _SP), _F32)
    s_i = _mm_tt_exact(row, ones_row)
    u = jax.lax.broadcasted_iota(jnp.int32, (_SP, _SP), 1)
    t = jax.lax.broadcasted_iota(jnp.int32, (_SP, _SP), 0)
    validu = (u >= 1) & (u <= _NP)
    rb = validu & ((s_j > s_i) | ((s_j == s_i) & (u < t)))
    rank = jnp.sum(rb.astype(_F32), axis=1, keepdims=True)
    tcol = jax.lax.broadcasted_iota(jnp.int32, (_SP, 1), 0)
    validt = (tcol >= 1) & (tcol <= _NP)
    keep = validt & (rank < float(_KK))
    eye = (u == t).astype(_F32)
    keep_row = jnp.broadcast_to(_mm_tt_exact(keep.astype(_F32), eye),
                                (_SP, _SP))
    eb = jnp.where(validu & (u < t), keep_row, 0.0)
    nbefore = jnp.sum(eb, axis=1, keepdims=True)
    pos = jnp.where(tcol == 0, 0.0,
                    jnp.where(keep, 1.0 + nbefore, -1.0))
    lanes = jax.lax.broadcasted_iota(jnp.int32, (_SP, _LP), 1).astype(_F32)
    onehot = (jnp.broadcast_to(pos, (_SP, _LP)) == lanes).astype(_F32)
    o_ref[0] = _mm_tt_exact(onehot, x)


def _pack_call(x, cls_attn):
    b = x.shape[0]
    return pl.pallas_call(
        _pack_body,
        grid=(b,),
        in_specs=[
            pl.BlockSpec((1, _SP, _C), lambda g: (g, 0, 0)),
            pl.BlockSpec((1, 1, _SP), lambda g: (g, 0, 0)),
        ],
        out_specs=pl.BlockSpec((1, _LP, _C), lambda g: (g, 0, 0)),
        out_shape=jax.ShapeDtypeStruct((b, _LP, _C), _F32),
    )(x, cls_attn)


def _head_body(c_ref, ns_ref, nb_ref, hw_ref, hb_ref, o_ref):
    cn = _ln(c_ref[...], ns_ref[...], nb_ref[...])
    o_ref[...] = _mm(cn, hw_ref[...]) + hb_ref[...]


def kernel(images, patch_w, patch_b, cls_tok, pos, qkv_w, qkv_b, proj_w,
           proj_b, ln1_s, ln1_b, ln2_s, ln2_b, mlp_w1, mlp_b1, mlp_w2,
           mlp_b2, norm_s, norm_b, head_w, head_b):
    b = images.shape[0]
    patches = images.reshape(b, 3, _G, _P, _G, _P).transpose(
        0, 2, 4, 1, 3, 5).reshape(b, _NP, _PD)
    p2 = jnp.concatenate(
        [jnp.zeros((b, 1, _PD), _F32), patches,
         jnp.zeros((b, _SP - _S, _PD), _F32)], axis=1).reshape(b * _SP, _PD)
    amap = jnp.concatenate(
        [cls_tok[0, 0:1] + pos[0, 0:1],
         pos[0, 1:] + patch_b[None, :],
         jnp.zeros((_SP - _S, _C), _F32)], axis=0)
    amap_full = jnp.tile(amap, (b, 1))

    x = pl.pallas_call(
        _embed_body,
        in_specs=[_full2d((b * _SP, _PD)), _full2d((_PD, _C)),
                  _full2d((b * _SP, _C))],
        out_specs=_full2d((b * _SP, _C)),
        out_shape=jax.ShapeDtypeStruct((b * _SP, _C), _F32),
    )(p2, patch_w, amap_full)

    wargs = (qkv_w, qkv_b, proj_w, proj_b, ln1_s, ln1_b, ln2_s, ln2_b,
             mlp_w1, mlp_b1, mlp_w2, mlp_b2)
    cls_attn = None
    for i in range(_DEPTH_EARLY):
        capture = i == _DEPTH_EARLY - 1
        res = _block(x, i, *wargs, sp=_SP, valid=_S, capture=capture)
        if capture:
            x, cls_attn = res
        else:
            x = res

    packed = _pack_call(x.reshape(b, _SP, _C), cls_attn)
    packed = packed.reshape(b * _LP, _C)

    for i in range(_DEPTH_EARLY, _DEPTH):
        packed = _block(packed, i, *wargs, sp=_LP, valid=_L, capture=False)

    cls_rows = packed.reshape(b, _LP, _C)[:, 0, :]
    out = pl.pallas_call(
        _head_body,
        in_specs=[_full2d((b, _C)), _full2d((1, _C)), _full2d((1, _C)),
                  _full2d((_C, _NCLS)), _full2d((1, _NCLS))],
        out_specs=_full2d((b, _NCLS)),
        out_shape=jax.ShapeDtypeStruct((b, _NCLS), _F32),
    )(cls_rows, norm_s.reshape(1, _C), norm_b.reshape(1, _C),
      head_w, head_b.reshape(1, _NCLS))
    return out


# speedup vs baseline: 1.0129x; 1.0129x over previous
import jax
import jax.numpy as jnp
from jax.experimental import pallas as pl

_B = 16
_IMG = 224
_P = 16
_C = 384
_H = 6
_DH = 64
_DEPTH_EARLY = 4
_DEPTH = 12
_NCLS = 1000
_G = _IMG // _P
_NP = _G * _G
_S = _NP + 1
_SP = 224
_KK = _S // 2
_L = _KK + 1
_LP = 128
_PD = 3 * _P * _P
_MT = 512

_F32 = jnp.float32
_BF16 = jnp.bfloat16


def _mm(a, b):
    return jax.lax.dot_general(a.astype(_BF16), b.astype(_BF16),
                               (((1,), (0,)), ((), ())),
                               preferred_element_type=_F32)


def _mm_t(a, b):
    return jax.lax.dot_general(a.astype(_BF16), b.astype(_BF16),
                               (((1,), (1,)), ((), ())),
                               preferred_element_type=_F32)


def _mm_tt_exact(a, b):
    return jax.lax.dot_general(a, b, (((0,), (0,)), ((), ())),
                               precision=jax.lax.Precision.HIGHEST,
                               preferred_element_type=_F32)


def _sum8tree(acc):
    s = acc[:, 0:8]
    for a in range(1, acc.shape[1] // 8):
        s = s + acc[:, 8 * a:8 * a + 8]
    t = s[:, 0:4] + s[:, 4:8]
    t = t[:, 0:2] + t[:, 2:4]
    return t[:, 0:1] + t[:, 1:2]


def _sum384(x):
    return _sum8tree((x[:, 0:128] + x[:, 128:256]) + x[:, 256:384])


def _ln(x, s, b):
    mu = _sum384(x) * (1.0 / 384.0)
    xc = x - mu
    v = _sum384(xc * xc) * (1.0 / 384.0)
    return xc * jax.lax.rsqrt(v + 1e-6) * s + b


def _softmax(lg):
    m = jnp.max(lg, axis=-1, keepdims=True)
    e = jnp.exp(lg - m)
    return e / _sum8tree(e)


def _full2d(shape):
    return pl.BlockSpec(shape, lambda *_: (0, 0))


def _embed_body(p_ref, a_ref, w_ref, o_ref):
    o_ref[...] = _mm(p_ref[...], w_ref[...]) + a_ref[...]


def _lnqkv_body(x_ref, s_ref, b_ref, w_ref, wb_ref, o_ref):
    xn = _ln(x_ref[...], s_ref[...], b_ref[...])
    o_ref[...] = _mm(xn, w_ref[...]) + wb_ref[...]


def _projmlp1_body(x_ref, o_ref, pw_ref, pb_ref, s_ref, b_ref, w1_ref,
                   b1_ref, x1_ref, h_ref):
    x1 = x_ref[...] + (_mm(o_ref[...], pw_ref[...]) + pb_ref[...])
    x1_ref[...] = x1
    xn2 = _ln(x1, s_ref[...], b_ref[...])
    h_ref[...] = jax.nn.gelu(_mm(xn2, w1_ref[...]) + b1_ref[...])


def _mlp2_body(x1_ref, h_ref, w2_ref, b2_ref, y_ref):
    y_ref[...] = x1_ref[...] + (_mm(h_ref[...], w2_ref[...]) + b2_ref[...])


def _rowtiled(body, m, ins, outs, n_rowtiled_in):
    in_specs = []
    args = []
    for idx, a in enumerate(ins):
        args.append(a)
        if idx < n_rowtiled_in:
            in_specs.append(pl.BlockSpec((_MT, a.shape[1]), lambda g: (g, 0)))
        else:
            in_specs.append(_full2d(a.shape))
    out_specs = [pl.BlockSpec((_MT, n), lambda g: (g, 0)) for n in outs]
    out_shape = [jax.ShapeDtypeStruct((m, n), _F32) for n in outs]
    if len(outs) == 1:
        out_specs, out_shape = out_specs[0], out_shape[0]
    return pl.pallas_call(body, grid=(m // _MT,), in_specs=in_specs,
                          out_specs=out_specs, out_shape=out_shape)(*args)


def _make_attn_body(sp, valid, capture):
    def body(qkv_ref, *out_refs):
        qkv = qkv_ref[...]
        lane = jax.lax.broadcasted_iota(jnp.int32, (sp, sp), 1)
        kmask = lane < valid
        scale = _DH ** -0.5
        outs = []
        cls_rows = []
        for h in range(_H):
            q = qkv[:, h * _DH:(h + 1) * _DH]
            k = qkv[:, _C + h * _DH:_C + (h + 1) * _DH]
            v = qkv[:, 2 * _C + h * _DH:2 * _C + (h + 1) * _DH]
            lg = _mm_t(q, k) * scale
            lg = jnp.where(kmask, lg, -1e30)
            a = _softmax(lg)
            if capture:
                cls_rows.append(a[0:1, :])
            outs.append(_mm(a, v))
        out_refs[0][...] = jnp.concatenate(outs, axis=1)
        if capture:
            out_refs[1][0] = jnp.mean(jnp.concatenate(cls_rows, axis=0),
                                      axis=0, keepdims=True)
    return body


def _attn_call(qkv, sp, valid, capture):
    b = qkv.shape[0] // sp
    out_specs = [pl.BlockSpec((sp, _C), lambda g: (g, 0))]
    out_shape = [jax.ShapeDtypeStruct((b * sp, _C), _F32)]
    if capture:
        out_specs.append(pl.BlockSpec((1, 1, sp), lambda g: (g, 0, 0)))
        out_shape.append(jax.ShapeDtypeStruct((b, 1, sp), _F32))
    res = pl.pallas_call(
        _make_attn_body(sp, valid, capture),
        grid=(b,),
        in_specs=[pl.BlockSpec((sp, 3 * _C), lambda g: (g, 0))],
        out_specs=out_specs,
        out_shape=out_shape,
    )(qkv)
    return res if capture else res[0]


def _block(x, i, qkv_w, qkv_b, proj_w, proj_b, ln1_s, ln1_b, ln2_s, ln2_b,
           mlp_w1, mlp_b1, mlp_w2, mlp_b2, sp, valid, capture):
    m = x.shape[0]
    qkv = _rowtiled(_lnqkv_body, m,
                    [x, ln1_s[i].reshape(1, _C), ln1_b[i].reshape(1, _C),
                     qkv_w[i], qkv_b[i].reshape(1, 3 * _C)],
                    [3 * _C], 1)
    ares = _attn_call(qkv, sp, valid, capture)
    if capture:
        o, cls_attn = ares
    else:
        o, cls_attn = ares, None
    x1, hid = _rowtiled(_projmlp1_body, m,
                        [x, o, proj_w[i], proj_b[i].reshape(1, _C),
                         ln2_s[i].reshape(1, _C), ln2_b[i].reshape(1, _C),
                         mlp_w1[i], mlp_b1[i].reshape(1, 4 * _C)],
                        [_C, 4 * _C], 2)
    y = _rowtiled(_mlp2_body, m,
                  [x1, hid, mlp_w2[i], mlp_b2[i].reshape(1, _C)],
                  [_C], 2)
    return (y, cls_attn) if capture else y


def _pack_body(x_ref, ca_ref, o_ref):
    x = x_ref[0]
    row = ca_ref[0]
    s_j = jnp.broadcast_to(row, (_SP, _SP))
    ones_row = jnp.ones((1, _SP), _F32)
    s_i = _mm_tt_exact(row, ones_row)
    u = jax.lax.broadcasted_iota(jnp.int32, (_SP, _SP), 1)
    t = jax.lax.broadcasted_iota(jnp.int32, (_SP, _SP), 0)
    validu = (u >= 1) & (u <= _NP)
    rb = validu & ((s_j > s_i) | ((s_j == s_i) & (u < t)))
    rank = jnp.sum(rb.astype(_F32), axis=1, keepdims=True)
    tcol = jax.lax.broadcasted_iota(jnp.int32, (_SP, 1), 0)
    validt = (tcol >= 1) & (tcol <= _NP)
    keep = validt & (rank < float(_KK))
    eye = (u == t).astype(_F32)
    keep_row = jnp.broadcast_to(_mm_tt_exact(keep.astype(_F32), eye),
                                (_SP, _SP))
    eb = jnp.where(validu & (u < t), keep_row, 0.0)
    nbefore = jnp.sum(eb, axis=1, keepdims=True)
    pos = jnp.where(tcol == 0, 0.0,
                    jnp.where(keep, 1.0 + nbefore, -1.0))
    lanes = jax.lax.broadcasted_iota(jnp.int32, (_SP, _LP), 1).astype(_F32)
    onehot = (jnp.broadcast_to(pos, (_SP, _LP)) == lanes).astype(_F32)
    o_ref[0] = _mm_tt_exact(onehot, x)


def _pack_call(x, cls_attn):
    b = x.shape[0]
    return pl.pallas_call(
        _pack_body,
        grid=(b,),
        in_specs=[
            pl.BlockSpec((1, _SP, _C), lambda g: (g, 0, 0)),
            pl.BlockSpec((1, 1, _SP), lambda g: (g, 0, 0)),
        ],
        out_specs=pl.BlockSpec((1, _LP, _C), lambda g: (g, 0, 0)),
        out_shape=jax.ShapeDtypeStruct((b, _LP, _C), _F32),
    )(x, cls_attn)


def _head_body(c_ref, ns_ref, nb_ref, hw_ref, hb_ref, o_ref):
    cn = _ln(c_ref[...], ns_ref[...], nb_ref[...])
    o_ref[...] = _mm(cn, hw_ref[...]) + hb_ref[...]


def kernel(images, patch_w, patch_b, cls_tok, pos, qkv_w, qkv_b, proj_w,
           proj_b, ln1_s, ln1_b, ln2_s, ln2_b, mlp_w1, mlp_b1, mlp_w2,
           mlp_b2, norm_s, norm_b, head_w, head_b):
    b = images.shape[0]
    patches = images.reshape(b, 3, _G, _P, _G, _P).transpose(
        0, 2, 4, 1, 3, 5).reshape(b, _NP, _PD)
    p2 = jnp.concatenate(
        [jnp.zeros((b, 1, _PD), _F32), patches,
         jnp.zeros((b, _SP - _S, _PD), _F32)], axis=1).reshape(b * _SP, _PD)
    amap = jnp.concatenate(
        [cls_tok[0, 0:1] + pos[0, 0:1],
         pos[0, 1:] + patch_b[None, :],
         jnp.zeros((_SP - _S, _C), _F32)], axis=0)
    amap_full = jnp.tile(amap, (b, 1))

    x = _rowtiled(_embed_body, b * _SP, [p2, amap_full, patch_w], [_C], 2)

    wargs = (qkv_w, qkv_b, proj_w, proj_b, ln1_s, ln1_b, ln2_s, ln2_b,
             mlp_w1, mlp_b1, mlp_w2, mlp_b2)
    cls_attn = None
    for i in range(_DEPTH_EARLY):
        capture = i == _DEPTH_EARLY - 1
        res = _block(x, i, *wargs, sp=_SP, valid=_S, capture=capture)
        if capture:
            x, cls_attn = res
        else:
            x = res

    packed = _pack_call(x.reshape(b, _SP, _C), cls_attn)
    packed = packed.reshape(b * _LP, _C)

    for i in range(_DEPTH_EARLY, _DEPTH):
        packed = _block(packed, i, *wargs, sp=_LP, valid=_L, capture=False)

    cls_rows = packed.reshape(b, _LP, _C)[:, 0, :]
    out = pl.pallas_call(
        _head_body,
        in_specs=[_full2d((b, _C)), _full2d((1, _C)), _full2d((1, _C)),
                  _full2d((_C, _NCLS)), _full2d((1, _NCLS))],
        out_specs=_full2d((b, _NCLS)),
        out_shape=jax.ShapeDtypeStruct((b, _NCLS), _F32),
    )(cls_rows, norm_s.reshape(1, _C), norm_b.reshape(1, _C),
      head_w, head_b.reshape(1, _NCLS))
    return out
